# baseline (speedup 1.0000x reference)
import jax
import jax.numpy as jnp
from jax import lax
from jax.experimental import pallas as pl
from jax.experimental.pallas import tpu as pltpu
from jax.experimental.pallas import tpu_sc as plsc

_NC = 2
_NS = 16
_NW = _NC * _NS
_CHUNK = 128


def _gather_body(idx_hbm, table_hbm, out_hbm, idx_v, rows_v, s0, s1):
    n_ch = idx_v.shape[0]
    wid = lax.axis_index("s") * _NC + lax.axis_index("c")
    base = wid * (n_ch * _CHUNK)
    sems = (s0, s1)

    pltpu.sync_copy(idx_hbm.at[wid], idx_v)

    pltpu.async_copy(table_hbm.at[idx_v.at[0]], rows_v.at[0], s0)
    pltpu.async_copy(table_hbm.at[idx_v.at[1]], rows_v.at[1], s1)

    def step(g, carry):
        for b in range(2):
            j = 2 * g + b
            pltpu.make_async_copy(
                table_hbm.at[pl.ds(0, _CHUNK)], rows_v.at[b], sems[b]
            ).wait()
            pltpu.sync_copy(
                rows_v.at[b], out_hbm.at[pl.ds(base + j * _CHUNK, _CHUNK)]
            )
            pltpu.async_copy(table_hbm.at[idx_v.at[j + 2]], rows_v.at[b], sems[b])
        return carry

    lax.fori_loop(0, n_ch // 2 - 1, step, 0)

    for b in range(2):
        j = n_ch - 2 + b
        pltpu.make_async_copy(
            table_hbm.at[pl.ds(0, _CHUNK)], rows_v.at[b], sems[b]
        ).wait()
        pltpu.sync_copy(
            rows_v.at[b], out_hbm.at[pl.ds(base + j * _CHUNK, _CHUNK)]
        )


def kernel(x, embed_weight):
    B, H = x.shape
    V, D = embed_weight.shape
    N = B * H
    n_per_w = N // _NW
    n_ch = n_per_w // _CHUNK
    idx = x.reshape(_NW, n_ch, _CHUNK).astype(jnp.int32)

    fn = pl.kernel(
        _gather_body,
        out_type=jax.ShapeDtypeStruct((N, D), jnp.float32),
        mesh=plsc.VectorSubcoreMesh(core_axis_name="c", subcore_axis_name="s"),
        compiler_params=pltpu.CompilerParams(use_tc_tiling_on_sc=False),
        scratch_types=[
            pltpu.VMEM((n_ch, _CHUNK), jnp.int32),
            pltpu.VMEM((2, _CHUNK, D), jnp.float32),
            pltpu.SemaphoreType.DMA,
            pltpu.SemaphoreType.DMA,
        ],
    )
    out = fn(idx, embed_weight)
    return out.reshape(B, H, D)
